# baseline (speedup 1.0000x reference)
import functools

import jax
import jax.numpy as jnp
from jax import lax
from jax.experimental import pallas as pl
from jax.experimental.pallas import tpu as pltpu
from jax.experimental.pallas import tpu_sc as plsc

N = 10000
E = 320000
NC = 2
NS = 16
NW = NC * NS
K = 128
EC = 2560
CPT = EC // NW
NB = 2
HC = CPT // 2
NGRP = HC // NB
NP = N + 8
RA = 624
RL = N - 15 * RA

_MESH = plsc.VectorSubcoreMesh(core_axis_name="c", subcore_axis_name="s")


def _make_prop(D):
    @functools.partial(
        pl.kernel,
        mesh=_MESH,
        out_type=jax.ShapeDtypeStruct((NC, N, D), jnp.float32),
        scratch_types=[
            pltpu.VMEM((HC, K), jnp.int32),
            pltpu.VMEM((HC, K), jnp.int32),
            pltpu.VMEM((NB, K, D), jnp.float32),
            pltpu.VMEM_SHARED((NP, D), jnp.float32),
        ] + [pltpu.SemaphoreType.DMA] * (2 * NB),
    )
    def prop(g_hbm, src_hbm, dst_hbm, out_hbm, srcv, dstv, rows, acc_sh, *sems):
        gsems = sems[:NB]
        ssems = sems[NB:]
        c = lax.axis_index("c")
        s = lax.axis_index("s")
        wid = c * NS + s
        r0 = s * RA

        @pl.when(s < NS - 1)
        def _():
            pltpu.sync_copy(g_hbm.at[pl.ds(r0, RA)], acc_sh.at[pl.ds(r0, RA)])

        @pl.when(s == NS - 1)
        def _():
            pltpu.sync_copy(g_hbm.at[pl.ds(r0, RL)], acc_sh.at[pl.ds(r0, RL)])

        plsc.subcore_barrier()

        def body(grp, carry):
            t0 = grp * NB
            ghs = [pltpu.async_copy(g_hbm.at[srcv.at[t0 + b]], rows.at[b],
                                    gsems[b])
                   for b in range(NB)]
            shs = []
            for b in range(NB):
                ghs[b].wait()
                shs.append(pltpu.async_copy(rows.at[b],
                                            acc_sh.at[dstv.at[t0 + b]],
                                            ssems[b], add=True))
            for b in range(NB):
                shs[b].wait()
            return carry

        for half in range(2):
            ch0 = wid * CPT + half * HC
            pltpu.sync_copy(src_hbm.at[pl.ds(ch0, HC)], srcv)
            pltpu.sync_copy(dst_hbm.at[pl.ds(ch0, HC)], dstv)
            lax.fori_loop(0, NGRP, body, 0)
        plsc.subcore_barrier()

        @pl.when(s < NS - 1)
        def _():
            pltpu.sync_copy(acc_sh.at[pl.ds(r0, RA)],
                            out_hbm.at[c].at[pl.ds(r0, RA)])

        @pl.when(s == NS - 1)
        def _():
            pltpu.sync_copy(acc_sh.at[pl.ds(r0, RL)],
                            out_hbm.at[c].at[pl.ds(r0, RL)])

    return prop


@functools.partial(
    pl.kernel,
    mesh=_MESH,
    out_type=jax.ShapeDtypeStruct((NC, NP), jnp.float32),
    scratch_types=[
        pltpu.VMEM((CPT, K), jnp.int32),
        pltpu.VMEM((K,), jnp.float32),
        pltpu.VMEM((NP,), jnp.float32),
        pltpu.VMEM_SHARED((NP,), jnp.float32),
    ] + [pltpu.SemaphoreType.DMA] * NB,
)
def _deg_kernel(dst_hbm, out_hbm, dstv, ones_v, z_v, acc_sh, *ssems):
    c = lax.axis_index("c")
    s = lax.axis_index("s")
    wid = c * NS + s
    pltpu.sync_copy(dst_hbm.at[pl.ds(wid * CPT, CPT)], dstv)

    def set_ones(i, carry):
        ones_v[pl.ds(i * 16, 16)] = jnp.ones((16,), jnp.float32)
        return carry

    lax.fori_loop(0, K // 16, set_ones, 0)

    @pl.when(s == 0)
    def _():
        def zero(i, carry):
            z_v[pl.ds(i * 16, 16)] = jnp.zeros((16,), jnp.float32)
            return carry

        lax.fori_loop(0, NP // 16, zero, 0)
        pltpu.sync_copy(z_v, acc_sh)

    plsc.subcore_barrier()

    def body(grp, carry):
        t0 = grp * NB
        shs = [pltpu.async_copy(ones_v, acc_sh.at[dstv.at[t0 + b]],
                                ssems[b], add=True)
               for b in range(NB)]
        for b in range(NB):
            shs[b].wait()
        return carry

    lax.fori_loop(0, CPT // NB, body, 0)
    plsc.subcore_barrier()

    @pl.when(s == 0)
    def _():
        pltpu.sync_copy(acc_sh, out_hbm.at[c])


_prop128 = _make_prop(128)


_R = 2000
_G = N // _R


def _t0_body(hist_ref, x_ref, w_ref, dis_ref, out_ref):
    deg = hist_ref[0] + hist_ref[1] + 1.0
    dis = lax.rsqrt(deg)
    dis_ref[...] = dis
    out_ref[...] = dis * jnp.dot(x_ref[...], w_ref[...],
                                 preferred_element_type=jnp.float32)


def _t0(hist, x, w):
    return pl.pallas_call(
        _t0_body,
        grid=(_G,),
        in_specs=[
            pl.BlockSpec((NC, _R, 1), lambda i: (0, i, 0)),
            pl.BlockSpec((_R, x.shape[1]), lambda i: (i, 0)),
            pl.BlockSpec(w.shape, lambda i: (0, 0)),
        ],
        out_specs=[
            pl.BlockSpec((_R, 1), lambda i: (i, 0)),
            pl.BlockSpec((_R, w.shape[1]), lambda i: (i, 0)),
        ],
        out_shape=[
            jax.ShapeDtypeStruct((N, 1), jnp.float32),
            jax.ShapeDtypeStruct((N, w.shape[1]), jnp.float32),
        ],
    )(hist, x, w)


def _ta_body(p_ref, g_ref, dis_ref, w_ref, out_ref):
    dis = dis_ref[...]
    q = p_ref[0] + p_ref[1] - g_ref[...]
    pre = jnp.maximum(dis * q, 0.0)
    out_ref[...] = dis * jnp.dot(pre, w_ref[...],
                                 preferred_element_type=jnp.float32)


def _ta(p, g, dis, w):
    d_in, d_out = w.shape
    return pl.pallas_call(
        _ta_body,
        grid=(_G,),
        in_specs=[
            pl.BlockSpec((NC, _R, d_in), lambda i: (0, i, 0)),
            pl.BlockSpec((_R, d_in), lambda i: (i, 0)),
            pl.BlockSpec((_R, 1), lambda i: (i, 0)),
            pl.BlockSpec(w.shape, lambda i: (0, 0)),
        ],
        out_specs=pl.BlockSpec((_R, d_out), lambda i: (i, 0)),
        out_shape=jax.ShapeDtypeStruct((N, d_out), jnp.float32),
    )(p, g, dis, w)


def _tb_body(p_ref, g_ref, dis_ref, out_ref):
    dis = dis_ref[...]
    q = p_ref[0] + p_ref[1] - g_ref[...]
    out_ref[...] = dis * dis * q


def _tb(p, g, dis):
    d = g.shape[1]
    return pl.pallas_call(
        _tb_body,
        grid=(_G,),
        in_specs=[
            pl.BlockSpec((NC, _R, d), lambda i: (0, i, 0)),
            pl.BlockSpec((_R, d), lambda i: (i, 0)),
            pl.BlockSpec((_R, 1), lambda i: (i, 0)),
        ],
        out_specs=pl.BlockSpec((_R, d), lambda i: (i, 0)),
        out_shape=jax.ShapeDtypeStruct((N, d), jnp.float32),
    )(p, g, dis)


def _tc_body(p_ref, g_ref, dis_ref, w_ref, b_ref, out_ref):
    dis = dis_ref[...]
    q = dis * (p_ref[0] + p_ref[1] - g_ref[...])
    h = jnp.dot(q, w_ref[...], preferred_element_type=jnp.float32) + b_ref[...]
    out_ref[...] = dis * jnp.maximum(h, 0.0)


def _tc(p, g, dis, w, b):
    d_in, d_out = w.shape
    return pl.pallas_call(
        _tc_body,
        grid=(_G,),
        in_specs=[
            pl.BlockSpec((NC, _R, d_in), lambda i: (0, i, 0)),
            pl.BlockSpec((_R, d_in), lambda i: (i, 0)),
            pl.BlockSpec((_R, 1), lambda i: (i, 0)),
            pl.BlockSpec(w.shape, lambda i: (0, 0)),
            pl.BlockSpec((1, d_out), lambda i: (0, 0)),
        ],
        out_specs=pl.BlockSpec((_R, d_out), lambda i: (i, 0)),
        out_shape=jax.ShapeDtypeStruct((N, d_out), jnp.float32),
    )(p, g, dis, w, b.reshape(1, -1))


def _td_body(p_ref, g_ref, dis_ref, w_ref, b_ref, out_ref):
    dis = dis_ref[...]
    q = dis * (p_ref[0] + p_ref[1] - g_ref[...])
    h = jnp.dot(q, w_ref[...], preferred_element_type=jnp.float32) + b_ref[...]
    m = jnp.max(h, axis=1, keepdims=True)
    e = jnp.exp(h - m)
    out_ref[...] = (h - m) - jnp.log(jnp.sum(e, axis=1, keepdims=True))


def _td(p, g, dis, w, b):
    d_in, d_out = w.shape
    return pl.pallas_call(
        _td_body,
        grid=(_G,),
        in_specs=[
            pl.BlockSpec((NC, _R, d_in), lambda i: (0, i, 0)),
            pl.BlockSpec((_R, d_in), lambda i: (i, 0)),
            pl.BlockSpec((_R, 1), lambda i: (i, 0)),
            pl.BlockSpec(w.shape, lambda i: (0, 0)),
            pl.BlockSpec((1, d_out), lambda i: (0, 0)),
        ],
        out_specs=pl.BlockSpec((_R, d_out), lambda i: (i, 0)),
        out_shape=jax.ShapeDtypeStruct((N, d_out), jnp.float32),
    )(p, g, dis, w, b.reshape(1, -1))


def kernel(x, edge_index, Wf1, Wf2, Wf3, W1, b1, W2, b2):
    ei = edge_index.astype(jnp.int32)
    pad = EC * K - E
    src = jnp.concatenate([ei[0], jnp.zeros((pad,), jnp.int32)]).reshape(EC, K)
    dst = jnp.concatenate([ei[1], jnp.full((pad,), N, jnp.int32)]).reshape(EC, K)

    Wf3p = jnp.pad(Wf3, ((0, 0), (0, 128 - Wf3.shape[1])))
    W1p = jnp.pad(W1, ((0, 128 - W1.shape[0]), (0, 0)))

    hist = _deg_kernel(dst)[:, :N]
    dis, g1 = _t0(hist.reshape(NC, N, 1), x, Wf1)

    p = _prop128(g1, src, dst)
    g2 = _ta(p, g1, dis, Wf2)
    p = _prop128(g2, src, dst)
    g3 = _ta(p, g2, dis, Wf3p)
    p = _prop128(g3, src, dst)
    g4 = _tb(p, g3, dis)
    p = _prop128(g4, src, dst)
    g5 = _tc(p, g4, dis, W1p, b1)
    p = _prop128(g5, src, dst)
    return _td(p, g5, dis, W2, b2)


# speedup vs baseline: 1.0018x; 1.0018x over previous
import functools

import jax
import jax.numpy as jnp
from jax import lax
from jax.experimental import pallas as pl
from jax.experimental.pallas import tpu as pltpu
from jax.experimental.pallas import tpu_sc as plsc

N = 10000
E = 320000
NC = 2
NS = 16
NW = NC * NS
K = 128
EC = 2560
CPT = EC // NW
NB = 2
HC = CPT // 2
NGRP = HC // NB
ND = 240
NP = N + ND
RA = 624
RL = N - 15 * RA

_MESH = plsc.VectorSubcoreMesh(core_axis_name="c", subcore_axis_name="s")


def _make_prop(D):
    @functools.partial(
        pl.kernel,
        mesh=_MESH,
        out_type=jax.ShapeDtypeStruct((NC, N, D), jnp.float32),
        scratch_types=[
            pltpu.VMEM((HC, K), jnp.int32),
            pltpu.VMEM((HC, K), jnp.int32),
            pltpu.VMEM((NB, K, D), jnp.float32),
            pltpu.VMEM_SHARED((NP, D), jnp.float32),
        ] + [pltpu.SemaphoreType.DMA] * (2 * NB),
    )
    def prop(g_hbm, src_hbm, dst_hbm, out_hbm, srcv, dstv, rows, acc_sh, *sems):
        gsems = sems[:NB]
        ssems = sems[NB:]
        c = lax.axis_index("c")
        s = lax.axis_index("s")
        wid = c * NS + s
        r0 = s * RA

        @pl.when(s < NS - 1)
        def _():
            pltpu.sync_copy(g_hbm.at[pl.ds(r0, RA)], acc_sh.at[pl.ds(r0, RA)])

        @pl.when(s == NS - 1)
        def _():
            pltpu.sync_copy(g_hbm.at[pl.ds(r0, RL)], acc_sh.at[pl.ds(r0, RL)])

        plsc.subcore_barrier()

        def body(grp, carry):
            t0 = grp * NB
            ghs = [pltpu.async_copy(g_hbm.at[srcv.at[t0 + b]], rows.at[b],
                                    gsems[b])
                   for b in range(NB)]
            shs = []
            for b in range(NB):
                ghs[b].wait()
                shs.append(pltpu.async_copy(rows.at[b],
                                            acc_sh.at[dstv.at[t0 + b]],
                                            ssems[b], add=True))
            for b in range(NB):
                shs[b].wait()
            return carry

        for half in range(2):
            ch0 = wid * CPT + half * HC
            pltpu.sync_copy(src_hbm.at[pl.ds(ch0, HC)], srcv)
            pltpu.sync_copy(dst_hbm.at[pl.ds(ch0, HC)], dstv)
            lax.fori_loop(0, NGRP, body, 0)
        plsc.subcore_barrier()

        @pl.when(s < NS - 1)
        def _():
            pltpu.sync_copy(acc_sh.at[pl.ds(r0, RA)],
                            out_hbm.at[c].at[pl.ds(r0, RA)])

        @pl.when(s == NS - 1)
        def _():
            pltpu.sync_copy(acc_sh.at[pl.ds(r0, RL)],
                            out_hbm.at[c].at[pl.ds(r0, RL)])

    return prop


@functools.partial(
    pl.kernel,
    mesh=_MESH,
    out_type=jax.ShapeDtypeStruct((NC, NP), jnp.float32),
    scratch_types=[
        pltpu.VMEM((CPT, K), jnp.int32),
        pltpu.VMEM((K,), jnp.float32),
        pltpu.VMEM((NP,), jnp.float32),
        pltpu.VMEM_SHARED((NP,), jnp.float32),
    ] + [pltpu.SemaphoreType.DMA] * NB,
)
def _deg_kernel(dst_hbm, out_hbm, dstv, ones_v, z_v, acc_sh, *ssems):
    c = lax.axis_index("c")
    s = lax.axis_index("s")
    wid = c * NS + s
    pltpu.sync_copy(dst_hbm.at[pl.ds(wid * CPT, CPT)], dstv)

    def set_ones(i, carry):
        ones_v[pl.ds(i * 16, 16)] = jnp.ones((16,), jnp.float32)
        return carry

    lax.fori_loop(0, K // 16, set_ones, 0)

    @pl.when(s == 0)
    def _():
        def zero(i, carry):
            z_v[pl.ds(i * 16, 16)] = jnp.zeros((16,), jnp.float32)
            return carry

        lax.fori_loop(0, NP // 16, zero, 0)
        pltpu.sync_copy(z_v, acc_sh)

    plsc.subcore_barrier()

    def body(grp, carry):
        t0 = grp * NB
        shs = [pltpu.async_copy(ones_v, acc_sh.at[dstv.at[t0 + b]],
                                ssems[b], add=True)
               for b in range(NB)]
        for b in range(NB):
            shs[b].wait()
        return carry

    lax.fori_loop(0, CPT // NB, body, 0)
    plsc.subcore_barrier()

    @pl.when(s == 0)
    def _():
        pltpu.sync_copy(acc_sh, out_hbm.at[c])


_prop128 = _make_prop(128)


_R = 2000
_G = N // _R


def _t0_body(hist_ref, x_ref, w_ref, dis_ref, out_ref):
    deg = hist_ref[0] + hist_ref[1] + 1.0
    dis = lax.rsqrt(deg)
    dis_ref[...] = dis
    out_ref[...] = dis * jnp.dot(x_ref[...], w_ref[...],
                                 preferred_element_type=jnp.float32)


def _t0(hist, x, w):
    return pl.pallas_call(
        _t0_body,
        grid=(_G,),
        in_specs=[
            pl.BlockSpec((NC, _R, 1), lambda i: (0, i, 0)),
            pl.BlockSpec((_R, x.shape[1]), lambda i: (i, 0)),
            pl.BlockSpec(w.shape, lambda i: (0, 0)),
        ],
        out_specs=[
            pl.BlockSpec((_R, 1), lambda i: (i, 0)),
            pl.BlockSpec((_R, w.shape[1]), lambda i: (i, 0)),
        ],
        out_shape=[
            jax.ShapeDtypeStruct((N, 1), jnp.float32),
            jax.ShapeDtypeStruct((N, w.shape[1]), jnp.float32),
        ],
    )(hist, x, w)


def _ta_body(p_ref, g_ref, dis_ref, w_ref, out_ref):
    dis = dis_ref[...]
    q = p_ref[0] + p_ref[1] - g_ref[...]
    pre = jnp.maximum(dis * q, 0.0)
    out_ref[...] = dis * jnp.dot(pre, w_ref[...],
                                 preferred_element_type=jnp.float32)


def _ta(p, g, dis, w):
    d_in, d_out = w.shape
    return pl.pallas_call(
        _ta_body,
        grid=(_G,),
        in_specs=[
            pl.BlockSpec((NC, _R, d_in), lambda i: (0, i, 0)),
            pl.BlockSpec((_R, d_in), lambda i: (i, 0)),
            pl.BlockSpec((_R, 1), lambda i: (i, 0)),
            pl.BlockSpec(w.shape, lambda i: (0, 0)),
        ],
        out_specs=pl.BlockSpec((_R, d_out), lambda i: (i, 0)),
        out_shape=jax.ShapeDtypeStruct((N, d_out), jnp.float32),
    )(p, g, dis, w)


def _tb_body(p_ref, g_ref, dis_ref, out_ref):
    dis = dis_ref[...]
    q = p_ref[0] + p_ref[1] - g_ref[...]
    out_ref[...] = dis * dis * q


def _tb(p, g, dis):
    d = g.shape[1]
    return pl.pallas_call(
        _tb_body,
        grid=(_G,),
        in_specs=[
            pl.BlockSpec((NC, _R, d), lambda i: (0, i, 0)),
            pl.BlockSpec((_R, d), lambda i: (i, 0)),
            pl.BlockSpec((_R, 1), lambda i: (i, 0)),
        ],
        out_specs=pl.BlockSpec((_R, d), lambda i: (i, 0)),
        out_shape=jax.ShapeDtypeStruct((N, d), jnp.float32),
    )(p, g, dis)


def _tc_body(p_ref, g_ref, dis_ref, w_ref, b_ref, out_ref):
    dis = dis_ref[...]
    q = dis * (p_ref[0] + p_ref[1] - g_ref[...])
    h = jnp.dot(q, w_ref[...], preferred_element_type=jnp.float32) + b_ref[...]
    out_ref[...] = dis * jnp.maximum(h, 0.0)


def _tc(p, g, dis, w, b):
    d_in, d_out = w.shape
    return pl.pallas_call(
        _tc_body,
        grid=(_G,),
        in_specs=[
            pl.BlockSpec((NC, _R, d_in), lambda i: (0, i, 0)),
            pl.BlockSpec((_R, d_in), lambda i: (i, 0)),
            pl.BlockSpec((_R, 1), lambda i: (i, 0)),
            pl.BlockSpec(w.shape, lambda i: (0, 0)),
            pl.BlockSpec((1, d_out), lambda i: (0, 0)),
        ],
        out_specs=pl.BlockSpec((_R, d_out), lambda i: (i, 0)),
        out_shape=jax.ShapeDtypeStruct((N, d_out), jnp.float32),
    )(p, g, dis, w, b.reshape(1, -1))


def _td_body(p_ref, g_ref, dis_ref, w_ref, b_ref, out_ref):
    dis = dis_ref[...]
    q = dis * (p_ref[0] + p_ref[1] - g_ref[...])
    h = jnp.dot(q, w_ref[...], preferred_element_type=jnp.float32) + b_ref[...]
    m = jnp.max(h, axis=1, keepdims=True)
    e = jnp.exp(h - m)
    out_ref[...] = (h - m) - jnp.log(jnp.sum(e, axis=1, keepdims=True))


def _td(p, g, dis, w, b):
    d_in, d_out = w.shape
    return pl.pallas_call(
        _td_body,
        grid=(_G,),
        in_specs=[
            pl.BlockSpec((NC, _R, d_in), lambda i: (0, i, 0)),
            pl.BlockSpec((_R, d_in), lambda i: (i, 0)),
            pl.BlockSpec((_R, 1), lambda i: (i, 0)),
            pl.BlockSpec(w.shape, lambda i: (0, 0)),
            pl.BlockSpec((1, d_out), lambda i: (0, 0)),
        ],
        out_specs=pl.BlockSpec((_R, d_out), lambda i: (i, 0)),
        out_shape=jax.ShapeDtypeStruct((N, d_out), jnp.float32),
    )(p, g, dis, w, b.reshape(1, -1))


def kernel(x, edge_index, Wf1, Wf2, Wf3, W1, b1, W2, b2):
    ei = edge_index.astype(jnp.int32)
    pad = EC * K - E
    pad_dst = N + (jnp.arange(pad, dtype=jnp.int32) % ND)
    src = jnp.concatenate([ei[0], jnp.zeros((pad,), jnp.int32)]).reshape(EC, K)
    dst = jnp.concatenate([ei[1], pad_dst]).reshape(EC, K)

    Wf3p = jnp.pad(Wf3, ((0, 0), (0, 128 - Wf3.shape[1])))
    W1p = jnp.pad(W1, ((0, 128 - W1.shape[0]), (0, 0)))

    hist = _deg_kernel(dst)[:, :N]
    dis, g1 = _t0(hist.reshape(NC, N, 1), x, Wf1)

    p = _prop128(g1, src, dst)
    g2 = _ta(p, g1, dis, Wf2)
    p = _prop128(g2, src, dst)
    g3 = _ta(p, g2, dis, Wf3p)
    p = _prop128(g3, src, dst)
    g4 = _tb(p, g3, dis)
    p = _prop128(g4, src, dst)
    g5 = _tc(p, g4, dis, W1p, b1)
    p = _prop128(g5, src, dst)
    return _td(p, g5, dis, W2, b2)


# speedup vs baseline: 1.0589x; 1.0570x over previous
import functools

import jax
import jax.numpy as jnp
from jax import lax
from jax.experimental import pallas as pl
from jax.experimental.pallas import tpu as pltpu
from jax.experimental.pallas import tpu_sc as plsc

N = 10000
E = 320000
NC = 2
NS = 16
NW = NC * NS
K = 128
EC = 2560
CPT = EC // NW
NB = 2
HC = CPT // 2
NGRP = HC // NB
ND = 240
NP = N + ND
RA = 624
RL = N - 15 * RA

_MESH = plsc.VectorSubcoreMesh(core_axis_name="c", subcore_axis_name="s")


def _make_prop(D):
    @functools.partial(
        pl.kernel,
        mesh=_MESH,
        out_type=jax.ShapeDtypeStruct((NC, N, D), jnp.float32),
        scratch_types=[
            pltpu.VMEM((HC, K), jnp.int32),
            pltpu.VMEM((HC, K), jnp.int32),
            pltpu.VMEM((NB, K, D), jnp.float32),
            pltpu.VMEM_SHARED((NP, D), jnp.float32),
        ] + [pltpu.SemaphoreType.DMA] * NB,
    )
    def prop(g_hbm, src_hbm, dst_hbm, out_hbm, srcv, dstv, rows, acc_sh, *sems):
        gsems = sems
        c = lax.axis_index("c")
        s = lax.axis_index("s")
        wid = c * NS + s
        r0 = s * RA

        @pl.when(s < NS - 1)
        def _():
            pltpu.sync_copy(g_hbm.at[pl.ds(r0, RA)], acc_sh.at[pl.ds(r0, RA)])

        @pl.when(s == NS - 1)
        def _():
            pltpu.sync_copy(g_hbm.at[pl.ds(r0, RL)], acc_sh.at[pl.ds(r0, RL)])

        plsc.subcore_barrier()

        def wait_gather(b):
            pltpu.make_async_copy(g_hbm.at[pl.ds(0, K)], rows.at[b],
                                  gsems[b]).wait()

        def body(j, carry):
            for b in range(NB):
                t = NB * j + b
                wait_gather(b)
                nxt = (b + 1) % NB

                @pl.when(t + 1 < HC)
                def _():
                    pltpu.async_copy(g_hbm.at[srcv.at[t + 1]], rows.at[nxt],
                                     gsems[nxt])

                pltpu.sync_copy(rows.at[b], acc_sh.at[dstv.at[t]], add=True)
            return carry

        for half in range(2):
            ch0 = wid * CPT + half * HC
            pltpu.sync_copy(src_hbm.at[pl.ds(ch0, HC)], srcv)
            pltpu.sync_copy(dst_hbm.at[pl.ds(ch0, HC)], dstv)
            pltpu.async_copy(g_hbm.at[srcv.at[0]], rows.at[0], gsems[0])
            lax.fori_loop(0, HC // NB, body, 0)
        plsc.subcore_barrier()

        @pl.when(s < NS - 1)
        def _():
            pltpu.sync_copy(acc_sh.at[pl.ds(r0, RA)],
                            out_hbm.at[c].at[pl.ds(r0, RA)])

        @pl.when(s == NS - 1)
        def _():
            pltpu.sync_copy(acc_sh.at[pl.ds(r0, RL)],
                            out_hbm.at[c].at[pl.ds(r0, RL)])

    return prop


@functools.partial(
    pl.kernel,
    mesh=_MESH,
    out_type=jax.ShapeDtypeStruct((NC, NP), jnp.float32),
    scratch_types=[
        pltpu.VMEM((CPT, K), jnp.int32),
        pltpu.VMEM((K,), jnp.float32),
        pltpu.VMEM((NP,), jnp.float32),
        pltpu.VMEM_SHARED((NP,), jnp.float32),
    ],
)
def _deg_kernel(dst_hbm, out_hbm, dstv, ones_v, z_v, acc_sh):
    c = lax.axis_index("c")
    s = lax.axis_index("s")
    wid = c * NS + s
    pltpu.sync_copy(dst_hbm.at[pl.ds(wid * CPT, CPT)], dstv)

    def set_ones(i, carry):
        ones_v[pl.ds(i * 16, 16)] = jnp.ones((16,), jnp.float32)
        return carry

    lax.fori_loop(0, K // 16, set_ones, 0)

    @pl.when(s == 0)
    def _():
        def zero(i, carry):
            z_v[pl.ds(i * 16, 16)] = jnp.zeros((16,), jnp.float32)
            return carry

        lax.fori_loop(0, NP // 16, zero, 0)
        pltpu.sync_copy(z_v, acc_sh)

    plsc.subcore_barrier()

    def body(t, carry):
        pltpu.sync_copy(ones_v, acc_sh.at[dstv.at[t]], add=True)
        return carry

    lax.fori_loop(0, CPT, body, 0)
    plsc.subcore_barrier()

    @pl.when(s == 0)
    def _():
        pltpu.sync_copy(acc_sh, out_hbm.at[c])


_prop128 = _make_prop(128)


_R = 2000
_G = N // _R


def _t0_body(hist_ref, x_ref, w_ref, dis_ref, out_ref):
    deg = hist_ref[0] + hist_ref[1] + 1.0
    dis = lax.rsqrt(deg)
    dis_ref[...] = dis
    out_ref[...] = dis * jnp.dot(x_ref[...], w_ref[...],
                                 preferred_element_type=jnp.float32)


def _t0(hist, x, w):
    return pl.pallas_call(
        _t0_body,
        grid=(_G,),
        in_specs=[
            pl.BlockSpec((NC, _R, 1), lambda i: (0, i, 0)),
            pl.BlockSpec((_R, x.shape[1]), lambda i: (i, 0)),
            pl.BlockSpec(w.shape, lambda i: (0, 0)),
        ],
        out_specs=[
            pl.BlockSpec((_R, 1), lambda i: (i, 0)),
            pl.BlockSpec((_R, w.shape[1]), lambda i: (i, 0)),
        ],
        out_shape=[
            jax.ShapeDtypeStruct((N, 1), jnp.float32),
            jax.ShapeDtypeStruct((N, w.shape[1]), jnp.float32),
        ],
    )(hist, x, w)


def _ta_body(p_ref, g_ref, dis_ref, w_ref, out_ref):
    dis = dis_ref[...]
    q = p_ref[0] + p_ref[1] - g_ref[...]
    pre = jnp.maximum(dis * q, 0.0)
    out_ref[...] = dis * jnp.dot(pre, w_ref[...],
                                 preferred_element_type=jnp.float32)


def _ta(p, g, dis, w):
    d_in, d_out = w.shape
    return pl.pallas_call(
        _ta_body,
        grid=(_G,),
        in_specs=[
            pl.BlockSpec((NC, _R, d_in), lambda i: (0, i, 0)),
            pl.BlockSpec((_R, d_in), lambda i: (i, 0)),
            pl.BlockSpec((_R, 1), lambda i: (i, 0)),
            pl.BlockSpec(w.shape, lambda i: (0, 0)),
        ],
        out_specs=pl.BlockSpec((_R, d_out), lambda i: (i, 0)),
        out_shape=jax.ShapeDtypeStruct((N, d_out), jnp.float32),
    )(p, g, dis, w)


def _tb_body(p_ref, g_ref, dis_ref, out_ref):
    dis = dis_ref[...]
    q = p_ref[0] + p_ref[1] - g_ref[...]
    out_ref[...] = dis * dis * q


def _tb(p, g, dis):
    d = g.shape[1]
    return pl.pallas_call(
        _tb_body,
        grid=(_G,),
        in_specs=[
            pl.BlockSpec((NC, _R, d), lambda i: (0, i, 0)),
            pl.BlockSpec((_R, d), lambda i: (i, 0)),
            pl.BlockSpec((_R, 1), lambda i: (i, 0)),
        ],
        out_specs=pl.BlockSpec((_R, d), lambda i: (i, 0)),
        out_shape=jax.ShapeDtypeStruct((N, d), jnp.float32),
    )(p, g, dis)


def _tc_body(p_ref, g_ref, dis_ref, w_ref, b_ref, out_ref):
    dis = dis_ref[...]
    q = dis * (p_ref[0] + p_ref[1] - g_ref[...])
    h = jnp.dot(q, w_ref[...], preferred_element_type=jnp.float32) + b_ref[...]
    out_ref[...] = dis * jnp.maximum(h, 0.0)


def _tc(p, g, dis, w, b):
    d_in, d_out = w.shape
    return pl.pallas_call(
        _tc_body,
        grid=(_G,),
        in_specs=[
            pl.BlockSpec((NC, _R, d_in), lambda i: (0, i, 0)),
            pl.BlockSpec((_R, d_in), lambda i: (i, 0)),
            pl.BlockSpec((_R, 1), lambda i: (i, 0)),
            pl.BlockSpec(w.shape, lambda i: (0, 0)),
            pl.BlockSpec((1, d_out), lambda i: (0, 0)),
        ],
        out_specs=pl.BlockSpec((_R, d_out), lambda i: (i, 0)),
        out_shape=jax.ShapeDtypeStruct((N, d_out), jnp.float32),
    )(p, g, dis, w, b.reshape(1, -1))


def _td_body(p_ref, g_ref, dis_ref, w_ref, b_ref, out_ref):
    dis = dis_ref[...]
    q = dis * (p_ref[0] + p_ref[1] - g_ref[...])
    h = jnp.dot(q, w_ref[...], preferred_element_type=jnp.float32) + b_ref[...]
    m = jnp.max(h, axis=1, keepdims=True)
    e = jnp.exp(h - m)
    out_ref[...] = (h - m) - jnp.log(jnp.sum(e, axis=1, keepdims=True))


def _td(p, g, dis, w, b):
    d_in, d_out = w.shape
    return pl.pallas_call(
        _td_body,
        grid=(_G,),
        in_specs=[
            pl.BlockSpec((NC, _R, d_in), lambda i: (0, i, 0)),
            pl.BlockSpec((_R, d_in), lambda i: (i, 0)),
            pl.BlockSpec((_R, 1), lambda i: (i, 0)),
            pl.BlockSpec(w.shape, lambda i: (0, 0)),
            pl.BlockSpec((1, d_out), lambda i: (0, 0)),
        ],
        out_specs=pl.BlockSpec((_R, d_out), lambda i: (i, 0)),
        out_shape=jax.ShapeDtypeStruct((N, d_out), jnp.float32),
    )(p, g, dis, w, b.reshape(1, -1))


def kernel(x, edge_index, Wf1, Wf2, Wf3, W1, b1, W2, b2):
    ei = edge_index.astype(jnp.int32)
    pad = EC * K - E
    pad_dst = N + (jnp.arange(pad, dtype=jnp.int32) % ND)
    src = jnp.concatenate([ei[0], jnp.zeros((pad,), jnp.int32)]).reshape(EC, K)
    dst = jnp.concatenate([ei[1], pad_dst]).reshape(EC, K)

    Wf3p = jnp.pad(Wf3, ((0, 0), (0, 128 - Wf3.shape[1])))
    W1p = jnp.pad(W1, ((0, 128 - W1.shape[0]), (0, 0)))

    hist = _deg_kernel(dst)[:, :N]
    dis, g1 = _t0(hist.reshape(NC, N, 1), x, Wf1)

    p = _prop128(g1, src, dst)
    g2 = _ta(p, g1, dis, Wf2)
    p = _prop128(g2, src, dst)
    g3 = _ta(p, g2, dis, Wf3p)
    p = _prop128(g3, src, dst)
    g4 = _tb(p, g3, dis)
    p = _prop128(g4, src, dst)
    g5 = _tc(p, g4, dis, W1p, b1)
    p = _prop128(g5, src, dst)
    return _td(p, g5, dis, W2, b2)


# speedup vs baseline: 2.3434x; 2.2130x over previous
import functools

import jax
import jax.numpy as jnp
from jax import lax
from jax.experimental import pallas as pl
from jax.experimental.pallas import tpu as pltpu
from jax.experimental.pallas import tpu_sc as plsc

N = 10000
E = 320000
NC = 2
NS = 16
NW = NC * NS
K = 128
NCHUNK = E // K
EC = 2560
CPT = EC // NW
NB = 2
HC = CPT // 2
NGRP = HC // NB
ND = 240
NP = N + ND
RA = 624
RL = N - 15 * RA

_MESH = plsc.VectorSubcoreMesh(core_axis_name="c", subcore_axis_name="s")


def _make_prop(D):
    @functools.partial(
        pl.kernel,
        mesh=_MESH,
        out_type=jax.ShapeDtypeStruct((NC, N, D), jnp.float32),
        scratch_types=[
            pltpu.VMEM((K,), jnp.int32),
            pltpu.VMEM((K,), jnp.int32),
            pltpu.VMEM((K,), jnp.int32),
            pltpu.VMEM((K, D), jnp.float32),
            pltpu.VMEM((K, D), jnp.float32),
            pltpu.VMEM_SHARED((NP, D), jnp.float32),
            pltpu.SemaphoreType.DMA,
            pltpu.SemaphoreType.DMA,
        ],
    )
    def prop(g_hbm, src_hbm, dst_hbm, out_hbm, srcv0, srcv1, dstv,
             rows0, rows1, acc_sh, gsem0, gsem1):
        c = lax.axis_index("c")
        s = lax.axis_index("s")
        wid = c * NS + s
        r0 = s * RA

        @pl.when(s < NS - 1)
        def _():
            pltpu.sync_copy(g_hbm.at[pl.ds(r0, RA)], acc_sh.at[pl.ds(r0, RA)])

        @pl.when(s == NS - 1)
        def _():
            pltpu.sync_copy(g_hbm.at[pl.ds(r0, RL)], acc_sh.at[pl.ds(r0, RL)])

        plsc.subcore_barrier()

        def body(j, carry):
            ch0 = wid + (2 * j) * NW
            ch1 = ch0 + NW

            @pl.when(ch0 < NCHUNK)
            def _():
                pltpu.sync_copy(src_hbm.at[pl.ds(ch0 * K, K)], srcv0)
                pltpu.async_copy(g_hbm.at[srcv0], rows0, gsem0)

            @pl.when(ch1 < NCHUNK)
            def _():
                pltpu.sync_copy(src_hbm.at[pl.ds(ch1 * K, K)], srcv1)
                pltpu.async_copy(g_hbm.at[srcv1], rows1, gsem1)

            @pl.when(ch0 < NCHUNK)
            def _():
                pltpu.make_async_copy(g_hbm.at[pl.ds(0, K)], rows0,
                                      gsem0).wait()
                pltpu.sync_copy(dst_hbm.at[pl.ds(ch0 * K, K)], dstv)
                pltpu.sync_copy(rows0, acc_sh.at[dstv], add=True)

            @pl.when(ch1 < NCHUNK)
            def _():
                pltpu.make_async_copy(g_hbm.at[pl.ds(0, K)], rows1,
                                      gsem1).wait()
                pltpu.sync_copy(dst_hbm.at[pl.ds(ch1 * K, K)], dstv)
                pltpu.sync_copy(rows1, acc_sh.at[dstv], add=True)

            return carry

        lax.fori_loop(0, (NCHUNK + 2 * NW - 1) // (2 * NW), body, 0)
        plsc.subcore_barrier()

        @pl.when(s < NS - 1)
        def _():
            pltpu.sync_copy(acc_sh.at[pl.ds(r0, RA)],
                            out_hbm.at[c].at[pl.ds(r0, RA)])

        @pl.when(s == NS - 1)
        def _():
            pltpu.sync_copy(acc_sh.at[pl.ds(r0, RL)],
                            out_hbm.at[c].at[pl.ds(r0, RL)])

    return prop


@functools.partial(
    pl.kernel,
    mesh=_MESH,
    out_type=jax.ShapeDtypeStruct((NC, NP), jnp.float32),
    scratch_types=[
        pltpu.VMEM((CPT, K), jnp.int32),
        pltpu.VMEM((K,), jnp.float32),
        pltpu.VMEM((NP,), jnp.float32),
        pltpu.VMEM_SHARED((NP,), jnp.float32),
    ],
)
def _deg_kernel(dst_hbm, out_hbm, dstv, ones_v, z_v, acc_sh):
    c = lax.axis_index("c")
    s = lax.axis_index("s")
    wid = c * NS + s
    pltpu.sync_copy(dst_hbm.at[pl.ds(wid * CPT, CPT)], dstv)

    def set_ones(i, carry):
        ones_v[pl.ds(i * 16, 16)] = jnp.ones((16,), jnp.float32)
        return carry

    lax.fori_loop(0, K // 16, set_ones, 0)

    @pl.when(s == 0)
    def _():
        def zero(i, carry):
            z_v[pl.ds(i * 16, 16)] = jnp.zeros((16,), jnp.float32)
            return carry

        lax.fori_loop(0, NP // 16, zero, 0)
        pltpu.sync_copy(z_v, acc_sh)

    plsc.subcore_barrier()

    def body(t, carry):
        pltpu.sync_copy(ones_v, acc_sh.at[dstv.at[t]], add=True)
        return carry

    lax.fori_loop(0, CPT, body, 0)
    plsc.subcore_barrier()

    @pl.when(s == 0)
    def _():
        pltpu.sync_copy(acc_sh, out_hbm.at[c])


_prop128 = _make_prop(128)


_R = 2000
_G = N // _R


def _t0_body(hist_ref, x_ref, w_ref, dis_ref, out_ref):
    deg = hist_ref[0] + hist_ref[1] + 1.0
    dis = lax.rsqrt(deg)
    dis_ref[...] = dis
    out_ref[...] = dis * jnp.dot(x_ref[...], w_ref[...],
                                 preferred_element_type=jnp.float32)


def _t0(hist, x, w):
    return pl.pallas_call(
        _t0_body,
        grid=(_G,),
        in_specs=[
            pl.BlockSpec((NC, _R, 1), lambda i: (0, i, 0)),
            pl.BlockSpec((_R, x.shape[1]), lambda i: (i, 0)),
            pl.BlockSpec(w.shape, lambda i: (0, 0)),
        ],
        out_specs=[
            pl.BlockSpec((_R, 1), lambda i: (i, 0)),
            pl.BlockSpec((_R, w.shape[1]), lambda i: (i, 0)),
        ],
        out_shape=[
            jax.ShapeDtypeStruct((N, 1), jnp.float32),
            jax.ShapeDtypeStruct((N, w.shape[1]), jnp.float32),
        ],
    )(hist, x, w)


def _ta_body(p_ref, g_ref, dis_ref, w_ref, out_ref):
    dis = dis_ref[...]
    q = p_ref[0] + p_ref[1] - g_ref[...]
    pre = jnp.maximum(dis * q, 0.0)
    out_ref[...] = dis * jnp.dot(pre, w_ref[...],
                                 preferred_element_type=jnp.float32)


def _ta(p, g, dis, w):
    d_in, d_out = w.shape
    return pl.pallas_call(
        _ta_body,
        grid=(_G,),
        in_specs=[
            pl.BlockSpec((NC, _R, d_in), lambda i: (0, i, 0)),
            pl.BlockSpec((_R, d_in), lambda i: (i, 0)),
            pl.BlockSpec((_R, 1), lambda i: (i, 0)),
            pl.BlockSpec(w.shape, lambda i: (0, 0)),
        ],
        out_specs=pl.BlockSpec((_R, d_out), lambda i: (i, 0)),
        out_shape=jax.ShapeDtypeStruct((N, d_out), jnp.float32),
    )(p, g, dis, w)


def _tb_body(p_ref, g_ref, dis_ref, out_ref):
    dis = dis_ref[...]
    q = p_ref[0] + p_ref[1] - g_ref[...]
    out_ref[...] = dis * dis * q


def _tb(p, g, dis):
    d = g.shape[1]
    return pl.pallas_call(
        _tb_body,
        grid=(_G,),
        in_specs=[
            pl.BlockSpec((NC, _R, d), lambda i: (0, i, 0)),
            pl.BlockSpec((_R, d), lambda i: (i, 0)),
            pl.BlockSpec((_R, 1), lambda i: (i, 0)),
        ],
        out_specs=pl.BlockSpec((_R, d), lambda i: (i, 0)),
        out_shape=jax.ShapeDtypeStruct((N, d), jnp.float32),
    )(p, g, dis)


def _tc_body(p_ref, g_ref, dis_ref, w_ref, b_ref, out_ref):
    dis = dis_ref[...]
    q = dis * (p_ref[0] + p_ref[1] - g_ref[...])
    h = jnp.dot(q, w_ref[...], preferred_element_type=jnp.float32) + b_ref[...]
    out_ref[...] = dis * jnp.maximum(h, 0.0)


def _tc(p, g, dis, w, b):
    d_in, d_out = w.shape
    return pl.pallas_call(
        _tc_body,
        grid=(_G,),
        in_specs=[
            pl.BlockSpec((NC, _R, d_in), lambda i: (0, i, 0)),
            pl.BlockSpec((_R, d_in), lambda i: (i, 0)),
            pl.BlockSpec((_R, 1), lambda i: (i, 0)),
            pl.BlockSpec(w.shape, lambda i: (0, 0)),
            pl.BlockSpec((1, d_out), lambda i: (0, 0)),
        ],
        out_specs=pl.BlockSpec((_R, d_out), lambda i: (i, 0)),
        out_shape=jax.ShapeDtypeStruct((N, d_out), jnp.float32),
    )(p, g, dis, w, b.reshape(1, -1))


def _td_body(p_ref, g_ref, dis_ref, w_ref, b_ref, out_ref):
    dis = dis_ref[...]
    q = dis * (p_ref[0] + p_ref[1] - g_ref[...])
    h = jnp.dot(q, w_ref[...], preferred_element_type=jnp.float32) + b_ref[...]
    m = jnp.max(h, axis=1, keepdims=True)
    e = jnp.exp(h - m)
    out_ref[...] = (h - m) - jnp.log(jnp.sum(e, axis=1, keepdims=True))


def _td(p, g, dis, w, b):
    d_in, d_out = w.shape
    return pl.pallas_call(
        _td_body,
        grid=(_G,),
        in_specs=[
            pl.BlockSpec((NC, _R, d_in), lambda i: (0, i, 0)),
            pl.BlockSpec((_R, d_in), lambda i: (i, 0)),
            pl.BlockSpec((_R, 1), lambda i: (i, 0)),
            pl.BlockSpec(w.shape, lambda i: (0, 0)),
            pl.BlockSpec((1, d_out), lambda i: (0, 0)),
        ],
        out_specs=pl.BlockSpec((_R, d_out), lambda i: (i, 0)),
        out_shape=jax.ShapeDtypeStruct((N, d_out), jnp.float32),
    )(p, g, dis, w, b.reshape(1, -1))


def kernel(x, edge_index, Wf1, Wf2, Wf3, W1, b1, W2, b2):
    ei = edge_index.astype(jnp.int32)
    src = ei[0]
    dst = ei[1]
    pad = EC * K - E
    pad_dst = N + (jnp.arange(pad, dtype=jnp.int32) % ND)
    dst2d = jnp.concatenate([dst, pad_dst]).reshape(EC, K)

    Wf3p = jnp.pad(Wf3, ((0, 0), (0, 128 - Wf3.shape[1])))
    W1p = jnp.pad(W1, ((0, 128 - W1.shape[0]), (0, 0)))

    hist = _deg_kernel(dst2d)[:, :N]
    dis, g1 = _t0(hist.reshape(NC, N, 1), x, Wf1)

    p = _prop128(g1, src, dst)
    g2 = _ta(p, g1, dis, Wf2)
    p = _prop128(g2, src, dst)
    g3 = _ta(p, g2, dis, Wf3p)
    p = _prop128(g3, src, dst)
    g4 = _tb(p, g3, dis)
    p = _prop128(g4, src, dst)
    g5 = _tc(p, g4, dis, W1p, b1)
    p = _prop128(g5, src, dst)
    return _td(p, g5, dis, W2, b2)


# speedup vs baseline: 2.7127x; 1.1576x over previous
import functools

import jax
import jax.numpy as jnp
from jax import lax
from jax.experimental import pallas as pl
from jax.experimental.pallas import tpu as pltpu
from jax.experimental.pallas import tpu_sc as plsc

N = 10000
E = 320000
NC = 2
NS = 16
NW = NC * NS
K = 128
NCHUNK = E // K
RA = 624
RL = N - 15 * RA

_MESH = plsc.VectorSubcoreMesh(core_axis_name="c", subcore_axis_name="s")


def _make_prop(D):
    nb = 3

    @functools.partial(
        pl.kernel,
        mesh=_MESH,
        out_type=jax.ShapeDtypeStruct((NC, N, D), jnp.float32),
        scratch_types=(
            [pltpu.VMEM((K,), jnp.int32)] * nb
            + [pltpu.VMEM((K,), jnp.int32)] * nb
            + [pltpu.VMEM((K, D), jnp.float32)] * nb
            + [pltpu.VMEM_SHARED((N, D), jnp.float32)]
            + [pltpu.SemaphoreType.DMA] * nb
        ),
    )
    def prop(g_hbm, src_hbm, dst_hbm, out_hbm, *refs):
        srcvs = refs[:nb]
        dstvs = refs[nb:2 * nb]
        rowss = refs[2 * nb:3 * nb]
        acc_sh = refs[3 * nb]
        gsems = refs[3 * nb + 1:]
        c = lax.axis_index("c")
        s = lax.axis_index("s")
        wid = c * NS + s
        r0 = s * RA

        @pl.when(s < NS - 1)
        def _():
            pltpu.sync_copy(g_hbm.at[pl.ds(r0, RA)], acc_sh.at[pl.ds(r0, RA)])

        @pl.when(s == NS - 1)
        def _():
            pltpu.sync_copy(g_hbm.at[pl.ds(r0, RL)], acc_sh.at[pl.ds(r0, RL)])

        plsc.subcore_barrier()

        def body(j, carry):
            chs = [wid + (nb * j + b) * NW for b in range(nb)]
            for b in range(nb):
                @pl.when(chs[b] < NCHUNK)
                def _(b=b):
                    pltpu.sync_copy(src_hbm.at[pl.ds(chs[b] * K, K)],
                                    srcvs[b])
                    pltpu.sync_copy(dst_hbm.at[pl.ds(chs[b] * K, K)],
                                    dstvs[b])
                    pltpu.async_copy(g_hbm.at[srcvs[b]], rowss[b], gsems[b])

            for b in range(nb):
                @pl.when(chs[b] < NCHUNK)
                def _(b=b):
                    pltpu.make_async_copy(g_hbm.at[pl.ds(0, K)], rowss[b],
                                          gsems[b]).wait()
                    pltpu.sync_copy(rowss[b], acc_sh.at[dstvs[b]], add=True)

            return carry

        lax.fori_loop(0, (NCHUNK + nb * NW - 1) // (nb * NW), body, 0)
        plsc.subcore_barrier()

        @pl.when(s < NS - 1)
        def _():
            pltpu.sync_copy(acc_sh.at[pl.ds(r0, RA)],
                            out_hbm.at[c].at[pl.ds(r0, RA)])

        @pl.when(s == NS - 1)
        def _():
            pltpu.sync_copy(acc_sh.at[pl.ds(r0, RL)],
                            out_hbm.at[c].at[pl.ds(r0, RL)])

    return prop


@functools.partial(
    pl.kernel,
    mesh=_MESH,
    out_type=jax.ShapeDtypeStruct((NC, N), jnp.float32),
    scratch_types=[
        pltpu.VMEM((K,), jnp.int32),
        pltpu.VMEM((K,), jnp.float32),
        pltpu.VMEM((N,), jnp.float32),
        pltpu.VMEM_SHARED((N,), jnp.float32),
    ],
)
def _deg_kernel(dst_hbm, out_hbm, dstv, ones_v, z_v, acc_sh):
    c = lax.axis_index("c")
    s = lax.axis_index("s")
    wid = c * NS + s

    def set_ones(i, carry):
        ones_v[pl.ds(i * 16, 16)] = jnp.ones((16,), jnp.float32)
        return carry

    lax.fori_loop(0, K // 16, set_ones, 0)

    @pl.when(s == 0)
    def _():
        def zero(i, carry):
            z_v[pl.ds(i * 16, 16)] = jnp.zeros((16,), jnp.float32)
            return carry

        lax.fori_loop(0, N // 16, zero, 0)
        pltpu.sync_copy(z_v, acc_sh)

    plsc.subcore_barrier()

    def body(t, carry):
        ch = wid + t * NW

        @pl.when(ch < NCHUNK)
        def _():
            pltpu.sync_copy(dst_hbm.at[pl.ds(ch * K, K)], dstv)
            pltpu.sync_copy(ones_v, acc_sh.at[dstv], add=True)

        return carry

    lax.fori_loop(0, (NCHUNK + NW - 1) // NW, body, 0)
    plsc.subcore_barrier()

    @pl.when(s == 0)
    def _():
        pltpu.sync_copy(acc_sh, out_hbm.at[c])


_prop128 = _make_prop(128)


_R = 2000
_G = N // _R


def _t0_body(hist_ref, x_ref, w_ref, dis_ref, out_ref):
    deg = hist_ref[0] + hist_ref[1] + 1.0
    dis = lax.rsqrt(deg)
    dis_ref[...] = dis
    out_ref[...] = dis * jnp.dot(x_ref[...], w_ref[...],
                                 preferred_element_type=jnp.float32)


def _t0(hist, x, w):
    return pl.pallas_call(
        _t0_body,
        grid=(_G,),
        in_specs=[
            pl.BlockSpec((NC, _R, 1), lambda i: (0, i, 0)),
            pl.BlockSpec((_R, x.shape[1]), lambda i: (i, 0)),
            pl.BlockSpec(w.shape, lambda i: (0, 0)),
        ],
        out_specs=[
            pl.BlockSpec((_R, 1), lambda i: (i, 0)),
            pl.BlockSpec((_R, w.shape[1]), lambda i: (i, 0)),
        ],
        out_shape=[
            jax.ShapeDtypeStruct((N, 1), jnp.float32),
            jax.ShapeDtypeStruct((N, w.shape[1]), jnp.float32),
        ],
    )(hist, x, w)


def _ta_body(p_ref, g_ref, dis_ref, w_ref, out_ref):
    dis = dis_ref[...]
    q = p_ref[0] + p_ref[1] - g_ref[...]
    pre = jnp.maximum(dis * q, 0.0)
    out_ref[...] = dis * jnp.dot(pre, w_ref[...],
                                 preferred_element_type=jnp.float32)


def _ta(p, g, dis, w):
    d_in, d_out = w.shape
    return pl.pallas_call(
        _ta_body,
        grid=(_G,),
        in_specs=[
            pl.BlockSpec((NC, _R, d_in), lambda i: (0, i, 0)),
            pl.BlockSpec((_R, d_in), lambda i: (i, 0)),
            pl.BlockSpec((_R, 1), lambda i: (i, 0)),
            pl.BlockSpec(w.shape, lambda i: (0, 0)),
        ],
        out_specs=pl.BlockSpec((_R, d_out), lambda i: (i, 0)),
        out_shape=jax.ShapeDtypeStruct((N, d_out), jnp.float32),
    )(p, g, dis, w)


def _tb_body(p_ref, g_ref, dis_ref, out_ref):
    dis = dis_ref[...]
    q = p_ref[0] + p_ref[1] - g_ref[...]
    out_ref[...] = dis * dis * q


def _tb(p, g, dis):
    d = g.shape[1]
    return pl.pallas_call(
        _tb_body,
        grid=(_G,),
        in_specs=[
            pl.BlockSpec((NC, _R, d), lambda i: (0, i, 0)),
            pl.BlockSpec((_R, d), lambda i: (i, 0)),
            pl.BlockSpec((_R, 1), lambda i: (i, 0)),
        ],
        out_specs=pl.BlockSpec((_R, d), lambda i: (i, 0)),
        out_shape=jax.ShapeDtypeStruct((N, d), jnp.float32),
    )(p, g, dis)


def _tc_body(p_ref, g_ref, dis_ref, w_ref, b_ref, out_ref):
    dis = dis_ref[...]
    q = dis * (p_ref[0] + p_ref[1] - g_ref[...])
    h = jnp.dot(q, w_ref[...], preferred_element_type=jnp.float32) + b_ref[...]
    out_ref[...] = dis * jnp.maximum(h, 0.0)


def _tc(p, g, dis, w, b):
    d_in, d_out = w.shape
    return pl.pallas_call(
        _tc_body,
        grid=(_G,),
        in_specs=[
            pl.BlockSpec((NC, _R, d_in), lambda i: (0, i, 0)),
            pl.BlockSpec((_R, d_in), lambda i: (i, 0)),
            pl.BlockSpec((_R, 1), lambda i: (i, 0)),
            pl.BlockSpec(w.shape, lambda i: (0, 0)),
            pl.BlockSpec((1, d_out), lambda i: (0, 0)),
        ],
        out_specs=pl.BlockSpec((_R, d_out), lambda i: (i, 0)),
        out_shape=jax.ShapeDtypeStruct((N, d_out), jnp.float32),
    )(p, g, dis, w, b.reshape(1, -1))


def _td_body(p_ref, g_ref, dis_ref, w_ref, b_ref, out_ref):
    dis = dis_ref[...]
    q = dis * (p_ref[0] + p_ref[1] - g_ref[...])
    h = jnp.dot(q, w_ref[...], preferred_element_type=jnp.float32) + b_ref[...]
    m = jnp.max(h, axis=1, keepdims=True)
    e = jnp.exp(h - m)
    out_ref[...] = (h - m) - jnp.log(jnp.sum(e, axis=1, keepdims=True))


def _td(p, g, dis, w, b):
    d_in, d_out = w.shape
    return pl.pallas_call(
        _td_body,
        grid=(_G,),
        in_specs=[
            pl.BlockSpec((NC, _R, d_in), lambda i: (0, i, 0)),
            pl.BlockSpec((_R, d_in), lambda i: (i, 0)),
            pl.BlockSpec((_R, 1), lambda i: (i, 0)),
            pl.BlockSpec(w.shape, lambda i: (0, 0)),
            pl.BlockSpec((1, d_out), lambda i: (0, 0)),
        ],
        out_specs=pl.BlockSpec((_R, d_out), lambda i: (i, 0)),
        out_shape=jax.ShapeDtypeStruct((N, d_out), jnp.float32),
    )(p, g, dis, w, b.reshape(1, -1))


def kernel(x, edge_index, Wf1, Wf2, Wf3, W1, b1, W2, b2):
    ei = edge_index.astype(jnp.int32)
    src = ei[0]
    dst = ei[1]

    Wf3p = jnp.pad(Wf3, ((0, 0), (0, 128 - Wf3.shape[1])))
    W1p = jnp.pad(W1, ((0, 128 - W1.shape[0]), (0, 0)))

    hist = _deg_kernel(dst)
    dis, g1 = _t0(hist.reshape(NC, N, 1), x, Wf1)

    p = _prop128(g1, src, dst)
    g2 = _ta(p, g1, dis, Wf2)
    p = _prop128(g2, src, dst)
    g3 = _ta(p, g2, dis, Wf3p)
    p = _prop128(g3, src, dst)
    g4 = _tb(p, g3, dis)
    p = _prop128(g4, src, dst)
    g5 = _tc(p, g4, dis, W1p, b1)
    p = _prop128(g5, src, dst)
    return _td(p, g5, dis, W2, b2)


# speedup vs baseline: 2.7480x; 1.0130x over previous
import functools

import jax
import jax.numpy as jnp
from jax import lax
from jax.experimental import pallas as pl
from jax.experimental.pallas import tpu as pltpu
from jax.experimental.pallas import tpu_sc as plsc

N = 10000
E = 320000
NC = 2
NS = 16
NW = NC * NS
K = 128
NCHUNK = E // K
RA = 624
RL = N - 15 * RA

_MESH = plsc.VectorSubcoreMesh(core_axis_name="c", subcore_axis_name="s")


def _make_prop(D):
    nb = 3

    @functools.partial(
        pl.kernel,
        mesh=_MESH,
        out_type=jax.ShapeDtypeStruct((NC, N, D), jnp.float32),
        scratch_types=(
            [pltpu.VMEM((K,), jnp.int32)] * nb
            + [pltpu.VMEM((K,), jnp.int32)] * nb
            + [pltpu.VMEM((K, D), jnp.float32)] * nb
            + [pltpu.VMEM_SHARED((N, D), jnp.float32)]
            + [pltpu.SemaphoreType.DMA] * (2 * nb)
        ),
    )
    def prop(g_hbm, src_hbm, dst_hbm, out_hbm, *refs):
        srcvs = refs[:nb]
        dstvs = refs[nb:2 * nb]
        rowss = refs[2 * nb:3 * nb]
        acc_sh = refs[3 * nb]
        gsems = refs[3 * nb + 1:3 * nb + 1 + nb]
        isems = refs[3 * nb + 1 + nb:]
        c = lax.axis_index("c")
        s = lax.axis_index("s")
        wid = c * NS + s
        r0 = s * RA

        @pl.when(s < NS - 1)
        def _():
            pltpu.sync_copy(g_hbm.at[pl.ds(r0, RA)], acc_sh.at[pl.ds(r0, RA)])

        @pl.when(s == NS - 1)
        def _():
            pltpu.sync_copy(g_hbm.at[pl.ds(r0, RL)], acc_sh.at[pl.ds(r0, RL)])

        plsc.subcore_barrier()

        def body(j, carry):
            chs = [wid + (nb * j + b) * NW for b in range(nb)]
            for b in range(nb):
                @pl.when(chs[b] < NCHUNK)
                def _(b=b):
                    pltpu.async_copy(src_hbm.at[pl.ds(chs[b] * K, K)],
                                    srcvs[b], isems[b])
                    pltpu.async_copy(dst_hbm.at[pl.ds(chs[b] * K, K)],
                                    dstvs[b], isems[b])

            for b in range(nb):
                @pl.when(chs[b] < NCHUNK)
                def _(b=b):
                    pltpu.make_async_copy(src_hbm.at[pl.ds(0, K)], srcvs[b],
                                          isems[b]).wait()
                    pltpu.make_async_copy(src_hbm.at[pl.ds(0, K)], dstvs[b],
                                          isems[b]).wait()
                    pltpu.async_copy(g_hbm.at[srcvs[b]], rowss[b], gsems[b])

            for b in range(nb):
                @pl.when(chs[b] < NCHUNK)
                def _(b=b):
                    pltpu.make_async_copy(g_hbm.at[pl.ds(0, K)], rowss[b],
                                          gsems[b]).wait()
                    pltpu.sync_copy(rowss[b], acc_sh.at[dstvs[b]], add=True)

            return carry

        lax.fori_loop(0, (NCHUNK + nb * NW - 1) // (nb * NW), body, 0)
        plsc.subcore_barrier()

        @pl.when(s < NS - 1)
        def _():
            pltpu.sync_copy(acc_sh.at[pl.ds(r0, RA)],
                            out_hbm.at[c].at[pl.ds(r0, RA)])

        @pl.when(s == NS - 1)
        def _():
            pltpu.sync_copy(acc_sh.at[pl.ds(r0, RL)],
                            out_hbm.at[c].at[pl.ds(r0, RL)])

    return prop


@functools.partial(
    pl.kernel,
    mesh=_MESH,
    out_type=jax.ShapeDtypeStruct((NC, N), jnp.float32),
    scratch_types=(
        [pltpu.VMEM((K,), jnp.int32)] * 6
        + [
            pltpu.VMEM((K,), jnp.float32),
            pltpu.VMEM((N,), jnp.float32),
            pltpu.VMEM_SHARED((N,), jnp.float32),
        ]
        + [pltpu.SemaphoreType.DMA] * 12
    ),
)
def _deg_kernel(dst_hbm, out_hbm, *refs):
    nbd = 6
    dstvs = refs[:nbd]
    ones_v, z_v, acc_sh = refs[nbd:nbd + 3]
    isems = refs[nbd + 3:nbd + 3 + nbd]
    ssems = refs[nbd + 3 + nbd:]
    c = lax.axis_index("c")
    s = lax.axis_index("s")
    wid = c * NS + s

    def set_ones(i, carry):
        ones_v[pl.ds(i * 16, 16)] = jnp.ones((16,), jnp.float32)
        return carry

    lax.fori_loop(0, K // 16, set_ones, 0)

    @pl.when(s == 0)
    def _():
        def zero(i, carry):
            z_v[pl.ds(i * 16, 16)] = jnp.zeros((16,), jnp.float32)
            return carry

        lax.fori_loop(0, N // 16, zero, 0)
        pltpu.sync_copy(z_v, acc_sh)

    plsc.subcore_barrier()

    def body(j, carry):
        chs = [wid + (nbd * j + b) * NW for b in range(nbd)]
        for b in range(nbd):
            @pl.when(chs[b] < NCHUNK)
            def _(b=b):
                pltpu.async_copy(dst_hbm.at[pl.ds(chs[b] * K, K)],
                                 dstvs[b], isems[b])

        for b in range(nbd):
            @pl.when(chs[b] < NCHUNK)
            def _(b=b):
                pltpu.make_async_copy(dst_hbm.at[pl.ds(0, K)], dstvs[b],
                                      isems[b]).wait()
                pltpu.async_copy(ones_v, acc_sh.at[dstvs[b]], ssems[b],
                                 add=True)

        for b in range(nbd):
            @pl.when(chs[b] < NCHUNK)
            def _(b=b):
                pltpu.make_async_copy(dst_hbm.at[pl.ds(0, K)], dstvs[b],
                                      ssems[b]).wait()

        return carry

    lax.fori_loop(0, (NCHUNK + nbd * NW - 1) // (nbd * NW), body, 0)
    plsc.subcore_barrier()

    @pl.when(s == 0)
    def _():
        pltpu.sync_copy(acc_sh, out_hbm.at[c])


_prop128 = _make_prop(128)


_R = 2000
_G = N // _R


def _t0_body(hist_ref, x_ref, w_ref, dis_ref, out_ref):
    deg = hist_ref[0] + hist_ref[1] + 1.0
    dis = lax.rsqrt(deg)
    dis_ref[...] = dis
    out_ref[...] = dis * jnp.dot(x_ref[...], w_ref[...],
                                 preferred_element_type=jnp.float32)


def _t0(hist, x, w):
    return pl.pallas_call(
        _t0_body,
        grid=(_G,),
        in_specs=[
            pl.BlockSpec((NC, _R, 1), lambda i: (0, i, 0)),
            pl.BlockSpec((_R, x.shape[1]), lambda i: (i, 0)),
            pl.BlockSpec(w.shape, lambda i: (0, 0)),
        ],
        out_specs=[
            pl.BlockSpec((_R, 1), lambda i: (i, 0)),
            pl.BlockSpec((_R, w.shape[1]), lambda i: (i, 0)),
        ],
        out_shape=[
            jax.ShapeDtypeStruct((N, 1), jnp.float32),
            jax.ShapeDtypeStruct((N, w.shape[1]), jnp.float32),
        ],
    )(hist, x, w)


def _ta_body(p_ref, g_ref, dis_ref, w_ref, out_ref):
    dis = dis_ref[...]
    q = p_ref[0] + p_ref[1] - g_ref[...]
    pre = jnp.maximum(dis * q, 0.0)
    out_ref[...] = dis * jnp.dot(pre, w_ref[...],
                                 preferred_element_type=jnp.float32)


def _ta(p, g, dis, w):
    d_in, d_out = w.shape
    return pl.pallas_call(
        _ta_body,
        grid=(_G,),
        in_specs=[
            pl.BlockSpec((NC, _R, d_in), lambda i: (0, i, 0)),
            pl.BlockSpec((_R, d_in), lambda i: (i, 0)),
            pl.BlockSpec((_R, 1), lambda i: (i, 0)),
            pl.BlockSpec(w.shape, lambda i: (0, 0)),
        ],
        out_specs=pl.BlockSpec((_R, d_out), lambda i: (i, 0)),
        out_shape=jax.ShapeDtypeStruct((N, d_out), jnp.float32),
    )(p, g, dis, w)


def _tb_body(p_ref, g_ref, dis_ref, out_ref):
    dis = dis_ref[...]
    q = p_ref[0] + p_ref[1] - g_ref[...]
    out_ref[...] = dis * dis * q


def _tb(p, g, dis):
    d = g.shape[1]
    return pl.pallas_call(
        _tb_body,
        grid=(_G,),
        in_specs=[
            pl.BlockSpec((NC, _R, d), lambda i: (0, i, 0)),
            pl.BlockSpec((_R, d), lambda i: (i, 0)),
            pl.BlockSpec((_R, 1), lambda i: (i, 0)),
        ],
        out_specs=pl.BlockSpec((_R, d), lambda i: (i, 0)),
        out_shape=jax.ShapeDtypeStruct((N, d), jnp.float32),
    )(p, g, dis)


def _tc_body(p_ref, g_ref, dis_ref, w_ref, b_ref, out_ref):
    dis = dis_ref[...]
    q = dis * (p_ref[0] + p_ref[1] - g_ref[...])
    h = jnp.dot(q, w_ref[...], preferred_element_type=jnp.float32) + b_ref[...]
    out_ref[...] = dis * jnp.maximum(h, 0.0)


def _tc(p, g, dis, w, b):
    d_in, d_out = w.shape
    return pl.pallas_call(
        _tc_body,
        grid=(_G,),
        in_specs=[
            pl.BlockSpec((NC, _R, d_in), lambda i: (0, i, 0)),
            pl.BlockSpec((_R, d_in), lambda i: (i, 0)),
            pl.BlockSpec((_R, 1), lambda i: (i, 0)),
            pl.BlockSpec(w.shape, lambda i: (0, 0)),
            pl.BlockSpec((1, d_out), lambda i: (0, 0)),
        ],
        out_specs=pl.BlockSpec((_R, d_out), lambda i: (i, 0)),
        out_shape=jax.ShapeDtypeStruct((N, d_out), jnp.float32),
    )(p, g, dis, w, b.reshape(1, -1))


def _td_body(p_ref, g_ref, dis_ref, w_ref, b_ref, out_ref):
    dis = dis_ref[...]
    q = dis * (p_ref[0] + p_ref[1] - g_ref[...])
    h = jnp.dot(q, w_ref[...], preferred_element_type=jnp.float32) + b_ref[...]
    m = jnp.max(h, axis=1, keepdims=True)
    e = jnp.exp(h - m)
    out_ref[...] = (h - m) - jnp.log(jnp.sum(e, axis=1, keepdims=True))


def _td(p, g, dis, w, b):
    d_in, d_out = w.shape
    return pl.pallas_call(
        _td_body,
        grid=(_G,),
        in_specs=[
            pl.BlockSpec((NC, _R, d_in), lambda i: (0, i, 0)),
            pl.BlockSpec((_R, d_in), lambda i: (i, 0)),
            pl.BlockSpec((_R, 1), lambda i: (i, 0)),
            pl.BlockSpec(w.shape, lambda i: (0, 0)),
            pl.BlockSpec((1, d_out), lambda i: (0, 0)),
        ],
        out_specs=pl.BlockSpec((_R, d_out), lambda i: (i, 0)),
        out_shape=jax.ShapeDtypeStruct((N, d_out), jnp.float32),
    )(p, g, dis, w, b.reshape(1, -1))


def kernel(x, edge_index, Wf1, Wf2, Wf3, W1, b1, W2, b2):
    ei = edge_index.astype(jnp.int32)
    src = ei[0]
    dst = ei[1]

    Wf3p = jnp.pad(Wf3, ((0, 0), (0, 128 - Wf3.shape[1])))
    W1p = jnp.pad(W1, ((0, 128 - W1.shape[0]), (0, 0)))

    hist = _deg_kernel(dst)
    dis, g1 = _t0(hist.reshape(NC, N, 1), x, Wf1)

    p = _prop128(g1, src, dst)
    g2 = _ta(p, g1, dis, Wf2)
    p = _prop128(g2, src, dst)
    g3 = _ta(p, g2, dis, Wf3p)
    p = _prop128(g3, src, dst)
    g4 = _tb(p, g3, dis)
    p = _prop128(g4, src, dst)
    g5 = _tc(p, g4, dis, W1p, b1)
    p = _prop128(g5, src, dst)
    return _td(p, g5, dis, W2, b2)


# speedup vs baseline: 2.7922x; 1.0161x over previous
import functools

import jax
import jax.numpy as jnp
from jax import lax
from jax.experimental import pallas as pl
from jax.experimental.pallas import tpu as pltpu
from jax.experimental.pallas import tpu_sc as plsc

N = 10000
E = 320000
NC = 2
NS = 16
NW = NC * NS
K = 128
NCHUNK = E // K
RA = 624
RL = N - 15 * RA

_MESH = plsc.VectorSubcoreMesh(core_axis_name="c", subcore_axis_name="s")


def _make_prop(D):
    nb = 3

    @functools.partial(
        pl.kernel,
        mesh=_MESH,
        out_type=jax.ShapeDtypeStruct((NC, N, D), jnp.float32),
        scratch_types=(
            [pltpu.VMEM((K,), jnp.int32)] * nb
            + [pltpu.VMEM((K,), jnp.int32)] * nb
            + [pltpu.VMEM((K, D), jnp.float32)] * nb
            + [pltpu.VMEM_SHARED((N, D), jnp.float32)]
            + [pltpu.SemaphoreType.DMA] * (3 * nb)
        ),
    )
    def prop(g_hbm, src_hbm, dst_hbm, out_hbm, *refs):
        srcvs = refs[:nb]
        dstvs = refs[nb:2 * nb]
        rowss = refs[2 * nb:3 * nb]
        acc_sh = refs[3 * nb]
        gsems = refs[3 * nb + 1:3 * nb + 1 + nb]
        isems = refs[3 * nb + 1 + nb:3 * nb + 1 + 2 * nb]
        ssems = refs[3 * nb + 1 + 2 * nb:]
        c = lax.axis_index("c")
        s = lax.axis_index("s")
        wid = c * NS + s
        r0 = s * RA

        @pl.when(s < NS - 1)
        def _():
            pltpu.sync_copy(g_hbm.at[pl.ds(r0, RA)], acc_sh.at[pl.ds(r0, RA)])

        @pl.when(s == NS - 1)
        def _():
            pltpu.sync_copy(g_hbm.at[pl.ds(r0, RL)], acc_sh.at[pl.ds(r0, RL)])

        plsc.subcore_barrier()

        def body(j, carry):
            chs = [wid + (nb * j + b) * NW for b in range(nb)]
            for b in range(nb):
                @pl.when(chs[b] < NCHUNK)
                def _(b=b):
                    pltpu.async_copy(src_hbm.at[pl.ds(chs[b] * K, K)],
                                    srcvs[b], isems[b])
                    pltpu.async_copy(dst_hbm.at[pl.ds(chs[b] * K, K)],
                                    dstvs[b], isems[b])

            for b in range(nb):
                @pl.when(chs[b] < NCHUNK)
                def _(b=b):
                    pltpu.make_async_copy(src_hbm.at[pl.ds(0, K)], srcvs[b],
                                          isems[b]).wait()
                    pltpu.make_async_copy(src_hbm.at[pl.ds(0, K)], dstvs[b],
                                          isems[b]).wait()
                    pltpu.async_copy(g_hbm.at[srcvs[b]], rowss[b], gsems[b])

            for b in range(nb):
                @pl.when(chs[b] < NCHUNK)
                def _(b=b):
                    pltpu.make_async_copy(g_hbm.at[pl.ds(0, K)], rowss[b],
                                          gsems[b]).wait()
                    pltpu.async_copy(rowss[b], acc_sh.at[dstvs[b]],
                                     ssems[b], add=True)

            for b in range(nb):
                @pl.when(chs[b] < NCHUNK)
                def _(b=b):
                    pltpu.make_async_copy(g_hbm.at[pl.ds(0, K)], rowss[b],
                                          ssems[b]).wait()

            return carry

        lax.fori_loop(0, (NCHUNK + nb * NW - 1) // (nb * NW), body, 0)
        plsc.subcore_barrier()

        @pl.when(s < NS - 1)
        def _():
            pltpu.sync_copy(acc_sh.at[pl.ds(r0, RA)],
                            out_hbm.at[c].at[pl.ds(r0, RA)])

        @pl.when(s == NS - 1)
        def _():
            pltpu.sync_copy(acc_sh.at[pl.ds(r0, RL)],
                            out_hbm.at[c].at[pl.ds(r0, RL)])

    return prop


@functools.partial(
    pl.kernel,
    mesh=_MESH,
    out_type=jax.ShapeDtypeStruct((NC, N), jnp.float32),
    scratch_types=(
        [pltpu.VMEM((K,), jnp.int32)] * 6
        + [
            pltpu.VMEM((K,), jnp.float32),
            pltpu.VMEM((N,), jnp.float32),
            pltpu.VMEM_SHARED((N,), jnp.float32),
        ]
        + [pltpu.SemaphoreType.DMA] * 12
    ),
)
def _deg_kernel(dst_hbm, out_hbm, *refs):
    nbd = 6
    dstvs = refs[:nbd]
    ones_v, z_v, acc_sh = refs[nbd:nbd + 3]
    isems = refs[nbd + 3:nbd + 3 + nbd]
    ssems = refs[nbd + 3 + nbd:]
    c = lax.axis_index("c")
    s = lax.axis_index("s")
    wid = c * NS + s

    def set_ones(i, carry):
        ones_v[pl.ds(i * 16, 16)] = jnp.ones((16,), jnp.float32)
        return carry

    lax.fori_loop(0, K // 16, set_ones, 0)

    @pl.when(s == 0)
    def _():
        def zero(i, carry):
            z_v[pl.ds(i * 16, 16)] = jnp.zeros((16,), jnp.float32)
            return carry

        lax.fori_loop(0, N // 16, zero, 0)
        pltpu.sync_copy(z_v, acc_sh)

    plsc.subcore_barrier()

    def body(j, carry):
        chs = [wid + (nbd * j + b) * NW for b in range(nbd)]
        for b in range(nbd):
            @pl.when(chs[b] < NCHUNK)
            def _(b=b):
                pltpu.async_copy(dst_hbm.at[pl.ds(chs[b] * K, K)],
                                 dstvs[b], isems[b])

        for b in range(nbd):
            @pl.when(chs[b] < NCHUNK)
            def _(b=b):
                pltpu.make_async_copy(dst_hbm.at[pl.ds(0, K)], dstvs[b],
                                      isems[b]).wait()
                pltpu.async_copy(ones_v, acc_sh.at[dstvs[b]], ssems[b],
                                 add=True)

        for b in range(nbd):
            @pl.when(chs[b] < NCHUNK)
            def _(b=b):
                pltpu.make_async_copy(dst_hbm.at[pl.ds(0, K)], dstvs[b],
                                      ssems[b]).wait()

        return carry

    lax.fori_loop(0, (NCHUNK + nbd * NW - 1) // (nbd * NW), body, 0)
    plsc.subcore_barrier()

    @pl.when(s == 0)
    def _():
        pltpu.sync_copy(acc_sh, out_hbm.at[c])


_prop128 = _make_prop(128)


_R = 2000
_G = N // _R


def _t0_body(hist_ref, x_ref, w_ref, dis_ref, out_ref):
    deg = hist_ref[0] + hist_ref[1] + 1.0
    dis = lax.rsqrt(deg)
    dis_ref[...] = dis
    out_ref[...] = dis * jnp.dot(x_ref[...], w_ref[...],
                                 preferred_element_type=jnp.float32)


def _t0(hist, x, w):
    return pl.pallas_call(
        _t0_body,
        grid=(_G,),
        in_specs=[
            pl.BlockSpec((NC, _R, 1), lambda i: (0, i, 0)),
            pl.BlockSpec((_R, x.shape[1]), lambda i: (i, 0)),
            pl.BlockSpec(w.shape, lambda i: (0, 0)),
        ],
        out_specs=[
            pl.BlockSpec((_R, 1), lambda i: (i, 0)),
            pl.BlockSpec((_R, w.shape[1]), lambda i: (i, 0)),
        ],
        out_shape=[
            jax.ShapeDtypeStruct((N, 1), jnp.float32),
            jax.ShapeDtypeStruct((N, w.shape[1]), jnp.float32),
        ],
    )(hist, x, w)


def _ta_body(p_ref, g_ref, dis_ref, w_ref, out_ref):
    dis = dis_ref[...]
    q = p_ref[0] + p_ref[1] - g_ref[...]
    pre = jnp.maximum(dis * q, 0.0)
    out_ref[...] = dis * jnp.dot(pre, w_ref[...],
                                 preferred_element_type=jnp.float32)


def _ta(p, g, dis, w):
    d_in, d_out = w.shape
    return pl.pallas_call(
        _ta_body,
        grid=(_G,),
        in_specs=[
            pl.BlockSpec((NC, _R, d_in), lambda i: (0, i, 0)),
            pl.BlockSpec((_R, d_in), lambda i: (i, 0)),
            pl.BlockSpec((_R, 1), lambda i: (i, 0)),
            pl.BlockSpec(w.shape, lambda i: (0, 0)),
        ],
        out_specs=pl.BlockSpec((_R, d_out), lambda i: (i, 0)),
        out_shape=jax.ShapeDtypeStruct((N, d_out), jnp.float32),
    )(p, g, dis, w)


def _tb_body(p_ref, g_ref, dis_ref, out_ref):
    dis = dis_ref[...]
    q = p_ref[0] + p_ref[1] - g_ref[...]
    out_ref[...] = dis * dis * q


def _tb(p, g, dis):
    d = g.shape[1]
    return pl.pallas_call(
        _tb_body,
        grid=(_G,),
        in_specs=[
            pl.BlockSpec((NC, _R, d), lambda i: (0, i, 0)),
            pl.BlockSpec((_R, d), lambda i: (i, 0)),
            pl.BlockSpec((_R, 1), lambda i: (i, 0)),
        ],
        out_specs=pl.BlockSpec((_R, d), lambda i: (i, 0)),
        out_shape=jax.ShapeDtypeStruct((N, d), jnp.float32),
    )(p, g, dis)


def _tc_body(p_ref, g_ref, dis_ref, w_ref, b_ref, out_ref):
    dis = dis_ref[...]
    q = dis * (p_ref[0] + p_ref[1] - g_ref[...])
    h = jnp.dot(q, w_ref[...], preferred_element_type=jnp.float32) + b_ref[...]
    out_ref[...] = dis * jnp.maximum(h, 0.0)


def _tc(p, g, dis, w, b):
    d_in, d_out = w.shape
    return pl.pallas_call(
        _tc_body,
        grid=(_G,),
        in_specs=[
            pl.BlockSpec((NC, _R, d_in), lambda i: (0, i, 0)),
            pl.BlockSpec((_R, d_in), lambda i: (i, 0)),
            pl.BlockSpec((_R, 1), lambda i: (i, 0)),
            pl.BlockSpec(w.shape, lambda i: (0, 0)),
            pl.BlockSpec((1, d_out), lambda i: (0, 0)),
        ],
        out_specs=pl.BlockSpec((_R, d_out), lambda i: (i, 0)),
        out_shape=jax.ShapeDtypeStruct((N, d_out), jnp.float32),
    )(p, g, dis, w, b.reshape(1, -1))


def _td_body(p_ref, g_ref, dis_ref, w_ref, b_ref, out_ref):
    dis = dis_ref[...]
    q = dis * (p_ref[0] + p_ref[1] - g_ref[...])
    h = jnp.dot(q, w_ref[...], preferred_element_type=jnp.float32) + b_ref[...]
    m = jnp.max(h, axis=1, keepdims=True)
    e = jnp.exp(h - m)
    out_ref[...] = (h - m) - jnp.log(jnp.sum(e, axis=1, keepdims=True))


def _td(p, g, dis, w, b):
    d_in, d_out = w.shape
    return pl.pallas_call(
        _td_body,
        grid=(_G,),
        in_specs=[
            pl.BlockSpec((NC, _R, d_in), lambda i: (0, i, 0)),
            pl.BlockSpec((_R, d_in), lambda i: (i, 0)),
            pl.BlockSpec((_R, 1), lambda i: (i, 0)),
            pl.BlockSpec(w.shape, lambda i: (0, 0)),
            pl.BlockSpec((1, d_out), lambda i: (0, 0)),
        ],
        out_specs=pl.BlockSpec((_R, d_out), lambda i: (i, 0)),
        out_shape=jax.ShapeDtypeStruct((N, d_out), jnp.float32),
    )(p, g, dis, w, b.reshape(1, -1))


def kernel(x, edge_index, Wf1, Wf2, Wf3, W1, b1, W2, b2):
    ei = edge_index.astype(jnp.int32)
    src = ei[0]
    dst = ei[1]

    Wf3p = jnp.pad(Wf3, ((0, 0), (0, 128 - Wf3.shape[1])))
    W1p = jnp.pad(W1, ((0, 128 - W1.shape[0]), (0, 0)))

    hist = _deg_kernel(dst)
    dis, g1 = _t0(hist.reshape(NC, N, 1), x, Wf1)

    p = _prop128(g1, src, dst)
    g2 = _ta(p, g1, dis, Wf2)
    p = _prop128(g2, src, dst)
    g3 = _ta(p, g2, dis, Wf3p)
    p = _prop128(g3, src, dst)
    g4 = _tb(p, g3, dis)
    p = _prop128(g4, src, dst)
    g5 = _tc(p, g4, dis, W1p, b1)
    p = _prop128(g5, src, dst)
    return _td(p, g5, dis, W2, b2)
